# baseline (speedup 1.0000x reference)
import ml_dtypes
import numpy as np
import jax
import jax.numpy as jnp
from jax.experimental import pallas as pl
from jax.experimental.pallas import tpu as pltpu

_N_OBJ = 256
_N_IM = 4
_C = 151
_H = 512
_OUT = 512
_D = 4096

_INV_C = np.float32(np.float32(1.0 / _C).astype(ml_dtypes.bfloat16))
_SLICES = 16
_NB = 10
_TAIL = _C - (_NB - 1) * _SLICES


def _b16(v):
    return v.astype(jnp.bfloat16)


def _bdot(a, b, dims=((1,), (1,))):
    return jax.lax.dot_general(_b16(a), _b16(b), (dims, ((), ())),
                               preferred_element_type=jnp.float32)


def _hdot(a, b, dims=((1,), (1,))):
    return jax.lax.dot_general(a, b, (dims, ((), ())),
                               preferred_element_type=jnp.float32,
                               precision=jax.lax.Precision.HIGHEST)


def _ggnn_kernel(im_inds_ref, obj_hbm, Wproj_hbm, bproj_ref,
                 W3w_hbm, b3w_ref, W3u_hbm, b3u_ref,
                 W4w_hbm, b4w_ref,
                 W5w_hbm, b5w_ref, W5u_hbm, b5u_ref,
                 Wout_hbm, bout_ref, Wcls_ref, bcls_ref,
                 out_ref,
                 obj_s, Wproj_s, W3w_s, W3u_s, W4w_s, W5w_s, W5u_s, Wout_s,
                 x_scr, h_scr, acc_scr,
                 sem_obj, sem_proj, sem_gates, sem_wout):
    c = pl.program_id(0)
    nc = pl.num_programs(0)

    cp_obj = pltpu.make_async_copy(obj_hbm, obj_s, sem_obj)
    cp_proj = pltpu.make_async_copy(Wproj_hbm, Wproj_s, sem_proj)
    cp_gates = [pltpu.make_async_copy(h, s, sem_gates)
                for h, s in ((W3w_hbm, W3w_s), (W3u_hbm, W3u_s),
                             (W4w_hbm, W4w_s), (W5w_hbm, W5w_s),
                             (W5u_hbm, W5u_s))]
    cp_wout = pltpu.make_async_copy(Wout_hbm, Wout_s, sem_wout)

    def gru_step(h):
        inds = im_inds_ref[...]
        hs = jnp.zeros_like(h)
        for im in range(_N_IM):
            m = inds == im
            s_im = jnp.sum(jnp.where(m, h, 0.0), axis=0, keepdims=True)
            hs = hs + jnp.where(m, s_im, 0.0)
        diff = hs - h
        a = jnp.float32(_C) * (_INV_C * _b16(diff).astype(jnp.float32))
        hU = _bdot(h, W3u_s[...]) + b3u_ref[0, :]
        zv = jax.nn.sigmoid(_bdot(a, W3w_s[:, :_H]) + _bdot(a, W3w_s[:, _H:])
                            + b3w_ref[0, :] + hU)
        rv = jax.nn.sigmoid(_bdot(a, W4w_s[:, :_H]) + _bdot(a, W4w_s[:, _H:])
                            + b4w_ref[0, :] + hU)
        hv = jnp.tanh(_bdot(a, W5w_s[:, :_H]) + _bdot(a, W5w_s[:, _H:])
                      + b5w_ref[0, :] + _bdot(rv * h, W5u_s[...]) + b5u_ref[0, :])
        return (1.0 - zv) * h + zv * hv

    @pl.when(c == 0)
    def _stage0():
        cp_obj.start()
        cp_proj.start()
        for cp in cp_gates:
            cp.start()
        cp_wout.start()
        cp_obj.wait()
        cp_proj.wait()
        x_scr[...] = _bdot(obj_s[...], Wproj_s[...]) + bproj_ref[0, :]

    @pl.when(c == 1)
    def _stage1():
        for cp in cp_gates:
            cp.wait()
        h_scr[...] = gru_step(x_scr[...])

    @pl.when((c == 2) | (c == 3))
    def _stage23():
        h_scr[...] = gru_step(h_scr[...])

    @pl.when(c == 0)
    def _init_acc():
        acc = Wcls_ref[:, :_OUT]
        for j in range(1, _SLICES):
            acc += Wcls_ref[:, j * _OUT:(j + 1) * _OUT]
        acc_scr[...] = acc

    @pl.when((c > 0) & (c < nc - 1))
    def _accum():
        acc = acc_scr[...]
        for j in range(_SLICES):
            acc += Wcls_ref[:, j * _OUT:(j + 1) * _OUT]
        acc_scr[...] = acc

    @pl.when(c == nc - 1)
    def _final():
        acc = acc_scr[...]
        for j in range(_TAIL):
            acc += Wcls_ref[:, j * _OUT:(j + 1) * _OUT]
        cp_wout.wait()
        o = _bdot(h_scr[...], Wout_s[:, :_H]) + _bdot(x_scr[...], Wout_s[:, _H:]) \
            + bout_ref[0, :]
        o = _b16(jnp.maximum(o, 0.0)).astype(jnp.float32)
        out_ref[...] = _hdot(o, acc) + bcls_ref[0, :]


def kernel(im_inds, obj_fmaps, obj_labels, Wproj, bproj, W3w, b3w, W3u, b3u,
           W4w, b4w, W4u, b4u, W5w, b5w, W5u, b5u, Wout, bout, Wcls, bcls):
    del obj_labels, W4u, b4u
    full = lambda shape: pl.BlockSpec(shape, lambda c: tuple(0 for _ in shape))
    hbm = lambda: pl.BlockSpec(memory_space=pl.ANY)
    row = lambda v: v.reshape(1, -1)
    return pl.pallas_call(
        _ggnn_kernel,
        grid=(_NB,),
        in_specs=[
            full((_N_OBJ, 1)),
            hbm(),
            hbm(),
            full((1, _H)),
            hbm(), full((1, _H)),
            hbm(), full((1, _H)),
            hbm(), full((1, _H)),
            hbm(), full((1, _H)),
            hbm(), full((1, _H)),
            hbm(), full((1, _OUT)),
            pl.BlockSpec((_C, _SLICES * _OUT), lambda c: (0, c)),
            full((1, _C)),
        ],
        out_specs=pl.BlockSpec((_N_OBJ, _C), lambda c: (0, 0)),
        out_shape=jax.ShapeDtypeStruct((_N_OBJ, _C), jnp.float32),
        scratch_shapes=[
            pltpu.VMEM((_N_OBJ, _D), jnp.float32),
            pltpu.VMEM((_H, _D), jnp.float32),
            pltpu.VMEM((_H, 2 * _H), jnp.float32),
            pltpu.VMEM((_H, _H), jnp.float32),
            pltpu.VMEM((_H, 2 * _H), jnp.float32),
            pltpu.VMEM((_H, 2 * _H), jnp.float32),
            pltpu.VMEM((_H, _H), jnp.float32),
            pltpu.VMEM((_OUT, 2 * _H), jnp.float32),
            pltpu.VMEM((_N_OBJ, _H), jnp.float32),
            pltpu.VMEM((_N_OBJ, _H), jnp.float32),
            pltpu.VMEM((_C, _OUT), jnp.float32),
            pltpu.SemaphoreType.DMA,
            pltpu.SemaphoreType.DMA,
            pltpu.SemaphoreType.DMA,
            pltpu.SemaphoreType.DMA,
        ],
    )(im_inds.reshape(_N_OBJ, 1), obj_fmaps, Wproj, row(bproj),
      W3w, row(b3w), W3u, row(b3u), W4w, row(b4w),
      W5w, row(b5w), W5u, row(b5u), Wout, row(bout), Wcls, row(bcls))


# speedup vs baseline: 1.0378x; 1.0378x over previous
import ml_dtypes
import numpy as np
import jax
import jax.numpy as jnp
from jax.experimental import pallas as pl
from jax.experimental.pallas import tpu as pltpu

_N_OBJ = 256
_N_IM = 4
_C = 151
_H = 512
_OUT = 512
_D = 4096

_INV_C = np.float32(np.float32(1.0 / _C).astype(ml_dtypes.bfloat16))
_SLICES = 8
_NB = 5


def _b16(v):
    return v.astype(jnp.bfloat16)


def _bdot(a, b, dims=((1,), (1,))):
    return jax.lax.dot_general(_b16(a), _b16(b), (dims, ((), ())),
                               preferred_element_type=jnp.float32)


def _hdot(a, b, dims=((1,), (1,))):
    return jax.lax.dot_general(a, b, (dims, ((), ())),
                               preferred_element_type=jnp.float32,
                               precision=jax.lax.Precision.HIGHEST)


def _ggnn_kernel(im_inds_ref, obj_hbm, Wproj_hbm, bproj_ref,
                 W3w_hbm, b3w_ref, W3u_hbm, b3u_ref,
                 W4w_hbm, b4w_ref,
                 W5w_hbm, b5w_ref, W5u_hbm, b5u_ref,
                 Wout_hbm, bout_ref, Wc0_ref, Wc1_ref, Wc2_ref, Wc3_ref,
                 bcls_ref,
                 out_ref,
                 obj_s, Wproj_s, W3w_s, W3u_s, W4w_s, W5w_s, W5u_s, Wout_s,
                 x_scr, h_scr, acc_scr,
                 sem_obj, sem_proj, sem_gates, sem_wout):
    c = pl.program_id(0)
    nc = pl.num_programs(0)

    cp_obj = pltpu.make_async_copy(obj_hbm, obj_s, sem_obj)
    cp_proj = pltpu.make_async_copy(Wproj_hbm, Wproj_s, sem_proj)
    cp_gates = [pltpu.make_async_copy(h, s, sem_gates)
                for h, s in ((W3w_hbm, W3w_s), (W3u_hbm, W3u_s),
                             (W4w_hbm, W4w_s), (W5w_hbm, W5w_s),
                             (W5u_hbm, W5u_s))]
    cp_wout = pltpu.make_async_copy(Wout_hbm, Wout_s, sem_wout)

    def gru_step(h):
        inds = im_inds_ref[...]
        hs = jnp.zeros_like(h)
        for im in range(_N_IM):
            m = inds == im
            s_im = jnp.sum(jnp.where(m, h, 0.0), axis=0, keepdims=True)
            hs = hs + jnp.where(m, s_im, 0.0)
        diff = hs - h
        a = jnp.float32(_C) * (_INV_C * _b16(diff).astype(jnp.float32))
        hU = _bdot(h, W3u_s[...]) + b3u_ref[0, :]
        zv = jax.nn.sigmoid(_bdot(a, W3w_s[:, :_H]) + _bdot(a, W3w_s[:, _H:])
                            + b3w_ref[0, :] + hU)
        rv = jax.nn.sigmoid(_bdot(a, W4w_s[:, :_H]) + _bdot(a, W4w_s[:, _H:])
                            + b4w_ref[0, :] + hU)
        hv = jnp.tanh(_bdot(a, W5w_s[:, :_H]) + _bdot(a, W5w_s[:, _H:])
                      + b5w_ref[0, :] + _bdot(rv * h, W5u_s[...]) + b5u_ref[0, :])
        return (1.0 - zv) * h + zv * hv

    @pl.when(c == 0)
    def _stage0():
        cp_obj.start()
        cp_proj.start()
        for cp in cp_gates:
            cp.start()
        cp_wout.start()
        cp_obj.wait()
        cp_proj.wait()
        x_scr[...] = _bdot(obj_s[...], Wproj_s[...]) + bproj_ref[0, :]

    @pl.when(c == 1)
    def _stage1():
        for cp in cp_gates:
            cp.wait()
        h_scr[...] = gru_step(x_scr[...])

    @pl.when((c == 2) | (c == 3))
    def _stage23():
        h_scr[...] = gru_step(h_scr[...])

    def _fold_streams(acc, refs_counts):
        for ref, count in refs_counts:
            for j in range(count):
                acc += ref[:, j * _OUT:(j + 1) * _OUT]
        return acc

    @pl.when(c == 0)
    def _init_acc():
        acc = Wc0_ref[:, :_OUT]
        for j in range(1, _SLICES):
            acc += Wc0_ref[:, j * _OUT:(j + 1) * _OUT]
        acc = _fold_streams(acc, [(Wc1_ref, _SLICES), (Wc2_ref, _SLICES),
                                  (Wc3_ref, _SLICES)])
        acc_scr[...] = acc

    @pl.when((c > 0) & (c < 3))
    def _accum():
        acc_scr[...] = _fold_streams(acc_scr[...],
                                     [(Wc0_ref, _SLICES), (Wc1_ref, _SLICES),
                                      (Wc2_ref, _SLICES), (Wc3_ref, _SLICES)])

    @pl.when(c == 3)
    def _accum_tail():
        acc_scr[...] = _fold_streams(acc_scr[...],
                                     [(Wc0_ref, _SLICES), (Wc1_ref, _SLICES),
                                      (Wc2_ref, _SLICES), (Wc3_ref, 7)])

    @pl.when(c == nc - 1)
    def _final():
        acc = _fold_streams(acc_scr[...],
                            [(Wc0_ref, _SLICES), (Wc1_ref, _SLICES),
                             (Wc2_ref, _SLICES)])
        cp_wout.wait()
        o = _bdot(h_scr[...], Wout_s[:, :_H]) + _bdot(x_scr[...], Wout_s[:, _H:]) \
            + bout_ref[0, :]
        o = _b16(jnp.maximum(o, 0.0)).astype(jnp.float32)
        out_ref[...] = _hdot(o, acc) + bcls_ref[0, :]


def kernel(im_inds, obj_fmaps, obj_labels, Wproj, bproj, W3w, b3w, W3u, b3u,
           W4w, b4w, W4u, b4u, W5w, b5w, W5u, b5u, Wout, bout, Wcls, bcls):
    del obj_labels, W4u, b4u
    full = lambda shape: pl.BlockSpec(shape, lambda c: tuple(0 for _ in shape))
    hbm = lambda: pl.BlockSpec(memory_space=pl.ANY)
    row = lambda v: v.reshape(1, -1)
    return pl.pallas_call(
        _ggnn_kernel,
        grid=(_NB,),
        in_specs=[
            full((_N_OBJ, 1)),
            hbm(),
            hbm(),
            full((1, _H)),
            hbm(), full((1, _H)),
            hbm(), full((1, _H)),
            hbm(), full((1, _H)),
            hbm(), full((1, _H)),
            hbm(), full((1, _H)),
            hbm(), full((1, _OUT)),
            pl.BlockSpec((_C, _SLICES * _OUT), lambda c: (0, c)),
            pl.BlockSpec((_C, _SLICES * _OUT), lambda c: (0, c + 5)),
            pl.BlockSpec((_C, _SLICES * _OUT), lambda c: (0, c + 10)),
            pl.BlockSpec((_C, _SLICES * _OUT),
                         lambda c: (0, jnp.minimum(c, 3) + 15)),
            full((1, _C)),
        ],
        out_specs=pl.BlockSpec((_N_OBJ, _C), lambda c: (0, 0)),
        out_shape=jax.ShapeDtypeStruct((_N_OBJ, _C), jnp.float32),
        scratch_shapes=[
            pltpu.VMEM((_N_OBJ, _D), jnp.float32),
            pltpu.VMEM((_H, _D), jnp.float32),
            pltpu.VMEM((_H, 2 * _H), jnp.float32),
            pltpu.VMEM((_H, _H), jnp.float32),
            pltpu.VMEM((_H, 2 * _H), jnp.float32),
            pltpu.VMEM((_H, 2 * _H), jnp.float32),
            pltpu.VMEM((_H, _H), jnp.float32),
            pltpu.VMEM((_OUT, 2 * _H), jnp.float32),
            pltpu.VMEM((_N_OBJ, _H), jnp.float32),
            pltpu.VMEM((_N_OBJ, _H), jnp.float32),
            pltpu.VMEM((_C, _OUT), jnp.float32),
            pltpu.SemaphoreType.DMA,
            pltpu.SemaphoreType.DMA,
            pltpu.SemaphoreType.DMA,
            pltpu.SemaphoreType.DMA,
        ],
    )(im_inds.reshape(_N_OBJ, 1), obj_fmaps, Wproj, row(bproj),
      W3w, row(b3w), W3u, row(b3u), W4w, row(b4w),
      W5w, row(b5w), W5u, row(b5u), Wout, row(bout),
      Wcls, Wcls, Wcls, Wcls, row(bcls))


# speedup vs baseline: 1.0488x; 1.0106x over previous
import ml_dtypes
import numpy as np
import jax
import jax.numpy as jnp
from jax.experimental import pallas as pl
from jax.experimental.pallas import tpu as pltpu

_N_OBJ = 256
_N_IM = 4
_C = 151
_H = 512
_OUT = 512
_D = 4096

_INV_C = np.float32(np.float32(1.0 / _C).astype(ml_dtypes.bfloat16))
_SLICES = 16
_NB = 5
_TAIL_B = _C - 80 - (_NB - 1) * _SLICES


def _b16(v):
    return v.astype(jnp.bfloat16)


def _bdot(a, b, dims=((1,), (1,))):
    return jax.lax.dot_general(_b16(a), _b16(b), (dims, ((), ())),
                               preferred_element_type=jnp.float32)


def _hdot(a, b, dims=((1,), (1,))):
    return jax.lax.dot_general(a, b, (dims, ((), ())),
                               preferred_element_type=jnp.float32,
                               precision=jax.lax.Precision.HIGHEST)


def _ggnn_kernel(im_inds_ref, obj_hbm, Wproj_hbm, bproj_ref,
                 W3w_hbm, b3w_ref, W3u_hbm, b3u_ref,
                 W4w_hbm, b4w_ref,
                 W5w_hbm, b5w_ref, W5u_hbm, b5u_ref,
                 Wout_hbm, bout_ref, WclsA_ref, WclsB_ref, bcls_ref,
                 out_ref,
                 obj_s, Wproj_s, W3w_s, W3u_s, W4w_s, W5w_s, W5u_s, Wout_s,
                 x_scr, h_scr, acc_scr,
                 sem_obj, sem_proj, sem_gates, sem_wout):
    c = pl.program_id(0)
    nc = pl.num_programs(0)

    cp_obj = pltpu.make_async_copy(obj_hbm, obj_s, sem_obj)
    cp_proj = pltpu.make_async_copy(Wproj_hbm, Wproj_s, sem_proj)
    cp_gates = [pltpu.make_async_copy(h, s, sem_gates)
                for h, s in ((W3w_hbm, W3w_s), (W3u_hbm, W3u_s),
                             (W4w_hbm, W4w_s), (W5w_hbm, W5w_s),
                             (W5u_hbm, W5u_s))]
    cp_wout = pltpu.make_async_copy(Wout_hbm, Wout_s, sem_wout)

    def gru_step(h):
        inds = im_inds_ref[...]
        hs = jnp.zeros_like(h)
        for im in range(_N_IM):
            m = inds == im
            s_im = jnp.sum(jnp.where(m, h, 0.0), axis=0, keepdims=True)
            hs = hs + jnp.where(m, s_im, 0.0)
        diff = hs - h
        a = jnp.float32(_C) * (_INV_C * _b16(diff).astype(jnp.float32))
        hU = _bdot(h, W3u_s[...]) + b3u_ref[0, :]
        zv = jax.nn.sigmoid(_bdot(a, W3w_s[:, :_H]) + _bdot(a, W3w_s[:, _H:])
                            + b3w_ref[0, :] + hU)
        rv = jax.nn.sigmoid(_bdot(a, W4w_s[:, :_H]) + _bdot(a, W4w_s[:, _H:])
                            + b4w_ref[0, :] + hU)
        hv = jnp.tanh(_bdot(a, W5w_s[:, :_H]) + _bdot(a, W5w_s[:, _H:])
                      + b5w_ref[0, :] + _bdot(rv * h, W5u_s[...]) + b5u_ref[0, :])
        return (1.0 - zv) * h + zv * hv

    @pl.when(c == 0)
    def _stage0():
        cp_obj.start()
        cp_proj.start()
        for cp in cp_gates:
            cp.start()
        cp_wout.start()
        cp_obj.wait()
        cp_proj.wait()
        x_scr[...] = _bdot(obj_s[...], Wproj_s[...]) + bproj_ref[0, :]

    @pl.when(c == 1)
    def _stage1():
        for cp in cp_gates:
            cp.wait()
        h_scr[...] = gru_step(x_scr[...])

    @pl.when((c == 2) | (c == 3))
    def _stage23():
        h_scr[...] = gru_step(h_scr[...])

    @pl.when(c == 0)
    def _init_acc():
        acc = WclsA_ref[:, :_OUT]
        for j in range(1, _SLICES):
            acc += WclsA_ref[:, j * _OUT:(j + 1) * _OUT]
        for j in range(_SLICES):
            acc += WclsB_ref[:, j * _OUT:(j + 1) * _OUT]
        acc_scr[...] = acc

    @pl.when((c > 0) & (c < nc - 1))
    def _accum():
        acc = acc_scr[...]
        for j in range(_SLICES):
            acc += WclsA_ref[:, j * _OUT:(j + 1) * _OUT]
        for j in range(_SLICES):
            acc += WclsB_ref[:, j * _OUT:(j + 1) * _OUT]
        acc_scr[...] = acc

    @pl.when(c == nc - 1)
    def _final():
        acc = acc_scr[...]
        for j in range(_SLICES):
            acc += WclsA_ref[:, j * _OUT:(j + 1) * _OUT]
        for j in range(_TAIL_B):
            acc += WclsB_ref[:, j * _OUT:(j + 1) * _OUT]
        cp_wout.wait()
        o = _bdot(h_scr[...], Wout_s[:, :_H]) + _bdot(x_scr[...], Wout_s[:, _H:]) \
            + bout_ref[0, :]
        o = _b16(jnp.maximum(o, 0.0)).astype(jnp.float32)
        out_ref[...] = _hdot(o, acc) + bcls_ref[0, :]


def kernel(im_inds, obj_fmaps, obj_labels, Wproj, bproj, W3w, b3w, W3u, b3u,
           W4w, b4w, W4u, b4u, W5w, b5w, W5u, b5u, Wout, bout, Wcls, bcls):
    del obj_labels, W4u, b4u
    full = lambda shape: pl.BlockSpec(shape, lambda c: tuple(0 for _ in shape))
    hbm = lambda: pl.BlockSpec(memory_space=pl.ANY)
    row = lambda v: v.reshape(1, -1)
    return pl.pallas_call(
        _ggnn_kernel,
        grid=(_NB,),
        in_specs=[
            full((_N_OBJ, 1)),
            hbm(),
            hbm(),
            full((1, _H)),
            hbm(), full((1, _H)),
            hbm(), full((1, _H)),
            hbm(), full((1, _H)),
            hbm(), full((1, _H)),
            hbm(), full((1, _H)),
            hbm(), full((1, _OUT)),
            pl.BlockSpec((_C, _SLICES * _OUT), lambda c: (0, c)),
            pl.BlockSpec((_C, _SLICES * _OUT), lambda c: (0, c + 5)),
            full((1, _C)),
        ],
        out_specs=pl.BlockSpec((_N_OBJ, _C), lambda c: (0, 0)),
        out_shape=jax.ShapeDtypeStruct((_N_OBJ, _C), jnp.float32),
        scratch_shapes=[
            pltpu.VMEM((_N_OBJ, _D), jnp.float32),
            pltpu.VMEM((_H, _D), jnp.float32),
            pltpu.VMEM((_H, 2 * _H), jnp.float32),
            pltpu.VMEM((_H, _H), jnp.float32),
            pltpu.VMEM((_H, 2 * _H), jnp.float32),
            pltpu.VMEM((_H, 2 * _H), jnp.float32),
            pltpu.VMEM((_H, _H), jnp.float32),
            pltpu.VMEM((_OUT, 2 * _H), jnp.float32),
            pltpu.VMEM((_N_OBJ, _H), jnp.float32),
            pltpu.VMEM((_N_OBJ, _H), jnp.float32),
            pltpu.VMEM((_C, _OUT), jnp.float32),
            pltpu.SemaphoreType.DMA,
            pltpu.SemaphoreType.DMA,
            pltpu.SemaphoreType.DMA,
            pltpu.SemaphoreType.DMA,
        ],
    )(im_inds.reshape(_N_OBJ, 1), obj_fmaps, Wproj, row(bproj),
      W3w, row(b3w), W3u, row(b3u), W4w, row(b4w),
      W5w, row(b5w), W5u, row(b5u), Wout, row(bout), Wcls, Wcls, row(bcls))
